# baseline (speedup 1.0000x reference)
import functools

import jax
import jax.numpy as jnp
from jax.experimental import pallas as pl
from jax.experimental.pallas import tpu as pltpu
from jax.experimental.pallas import tpu_sc as plsc
from jax.sharding import Mesh, PartitionSpec as P

try:
    from jax import shard_map as _shard_map
except ImportError:
    from jax.experimental.shard_map import shard_map as _shard_map

EMBED_DIM = 256
CODEBOOK = 8192
INPUT_DIM = 768
N_ROWS = 16 * 576
TILE = 256
GW = 128


def _vq_kernel(inp_ref, w_in_ref, b_in_ref, emb_ref,
               x_ref, enc_ref, idx_ref, emb_t_ref, emb_n_ref):
    @pl.when(pl.program_id(0) == 0)
    def _init():
        emb = emb_ref[...]
        norms = jnp.sqrt(jnp.sum(emb * emb, axis=0, keepdims=True))
        emb_n_ref[...] = (emb / (norms + 1e-12)).astype(jnp.bfloat16)
        emb_t_ref[...] = emb.T

    x = jnp.dot(inp_ref[...].astype(jnp.bfloat16),
                w_in_ref[...].astype(jnp.bfloat16),
                preferred_element_type=jnp.float32) + b_in_ref[...]
    x_ref[...] = x
    norm = jnp.sqrt(jnp.sum(x * x, axis=1, keepdims=True))
    x_n = x / (norm + 1e-12)
    logits = jnp.dot(x_n.astype(jnp.bfloat16), emb_n_ref[...],
                     preferred_element_type=jnp.float32)

    SLAB = CODEBOOK // 4
    iota_s = jax.lax.broadcasted_iota(jnp.int32, (TILE, SLAB), 1)

    def slab_argmax(s):
        sl = logits[:, s * SLAB:(s + 1) * SLAB]
        m = jnp.max(sl, axis=1, keepdims=True)
        j = jnp.min(jnp.where(sl == m, iota_s, CODEBOOK), axis=1) + s * SLAB
        return m[:, 0], j

    accv, accj = slab_argmax(0)
    accv = accv.astype(jnp.bfloat16).astype(jnp.float32)
    for s in range(1, 4):
        v, j = slab_argmax(s)
        upd = (v > accv) | ((v == accv) & (j < accj))
        accv = jnp.where(upd, v.astype(jnp.bfloat16).astype(jnp.float32), accv)
        accj = jnp.where(upd, j, accj)
    idx = accj

    iota = jax.lax.broadcasted_iota(jnp.int32, (TILE, CODEBOOK), 1)
    enc_ref[...] = (iota == idx[:, None]).astype(jnp.float32)
    idx_ref[0, 0, :] = idx


def _out_kernel(q_ref, x_ref, w_out_ref, b_out_ref, out_ref, loss_ref):
    q = q_ref[...]
    out_ref[...] = jnp.dot(q.astype(jnp.bfloat16),
                           w_out_ref[...].astype(jnp.bfloat16),
                           preferred_element_type=jnp.float32) + b_out_ref[...]
    diff = q - x_ref[...]
    part = 2.0 * jnp.sum(diff * diff) / (N_ROWS * EMBED_DIM)
    loss_ref[...] = part.reshape(1, 1, 1)


def _sc_gather(emb_t, idx_flat):
    n_rows = idx_flat.shape[1]
    mesh = plsc.VectorSubcoreMesh(core_axis_name="core",
                                  subcore_axis_name="subcore")

    @functools.partial(
        pl.kernel,
        out_type=jax.ShapeDtypeStruct((n_rows, EMBED_DIM), jnp.float32),
        mesh=mesh)
    def gather_kernel(t_hbm, i_hbm, q_hbm):
        def body(i_vmem, o_vmem):
            pltpu.sync_copy(t_hbm.at[i_vmem.at[0]], o_vmem)

        pltpu.emit_pipeline(
            body,
            grid=(n_rows // GW,),
            in_specs=[pl.BlockSpec((1, GW), index_map=lambda i: (0, i))],
            out_specs=[pl.BlockSpec((GW, EMBED_DIM), index_map=lambda i: (i, 0))],
            core_axis_name="subcore",
            dimension_semantics=(pltpu.PARALLEL,),
        )(i_hbm, q_hbm)

    return gather_kernel(emb_t, idx_flat)


def _shard_body(inp_flat, W_in, b_in, embeddings, W_out, b_out):
    n_rows = inp_flat.shape[0]
    n_tiles = n_rows // TILE

    x, enc, idx, emb_t = pl.pallas_call(
        _vq_kernel,
        grid=(n_tiles,),
        in_specs=[
            pl.BlockSpec((TILE, INPUT_DIM), lambda i: (i, 0)),
            pl.BlockSpec((INPUT_DIM, EMBED_DIM), lambda i: (0, 0)),
            pl.BlockSpec((1, EMBED_DIM), lambda i: (0, 0)),
            pl.BlockSpec((EMBED_DIM, CODEBOOK), lambda i: (0, 0)),
        ],
        out_specs=[
            pl.BlockSpec((TILE, EMBED_DIM), lambda i: (i, 0)),
            pl.BlockSpec((TILE, CODEBOOK), lambda i: (i, 0)),
            pl.BlockSpec((1, 1, TILE), lambda i: (i, 0, 0)),
            pl.BlockSpec((CODEBOOK, EMBED_DIM), lambda i: (0, 0)),
        ],
        out_shape=[
            jax.ShapeDtypeStruct((n_rows, EMBED_DIM), jnp.float32),
            jax.ShapeDtypeStruct((n_rows, CODEBOOK), jnp.float32),
            jax.ShapeDtypeStruct((n_tiles, 1, TILE), jnp.int32),
            jax.ShapeDtypeStruct((CODEBOOK, EMBED_DIM), jnp.float32),
        ],
        scratch_shapes=[pltpu.VMEM((EMBED_DIM, CODEBOOK), jnp.bfloat16)],
        compiler_params=pltpu.CompilerParams(
            dimension_semantics=("arbitrary",)),
    )(inp_flat, W_in, b_in, embeddings)

    q = _sc_gather(emb_t, idx.reshape(1, n_rows))

    out, loss = pl.pallas_call(
        _out_kernel,
        grid=(n_tiles,),
        in_specs=[
            pl.BlockSpec((TILE, EMBED_DIM), lambda i: (i, 0)),
            pl.BlockSpec((TILE, EMBED_DIM), lambda i: (i, 0)),
            pl.BlockSpec((EMBED_DIM, INPUT_DIM), lambda i: (0, 0)),
            pl.BlockSpec((1, INPUT_DIM), lambda i: (0, 0)),
        ],
        out_specs=[
            pl.BlockSpec((TILE, INPUT_DIM), lambda i: (i, 0)),
            pl.BlockSpec((1, 1, 1), lambda i: (i, 0, 0)),
        ],
        out_shape=[
            jax.ShapeDtypeStruct((n_rows, INPUT_DIM), jnp.float32),
            jax.ShapeDtypeStruct((n_tiles, 1, 1), jnp.float32),
        ],
        compiler_params=pltpu.CompilerParams(
            dimension_semantics=("arbitrary",)),
    )(q, x, W_out, b_out)

    return out, enc, idx, loss


@jax.jit
def kernel(inputs, W_in, b_in, embeddings, W_out, b_out):
    B, T, _ = inputs.shape
    inp_flat = inputs.reshape(N_ROWS, INPUT_DIM)
    b_in2 = b_in.reshape(1, EMBED_DIM)
    b_out2 = b_out.reshape(1, INPUT_DIM)

    devs = [d for d in jax.devices() if d.platform == "tpu"][:2]
    if len(devs) == 2:
        mesh = Mesh(devs, ("x",))
        sharded = _shard_map(
            _shard_body, mesh=mesh,
            in_specs=(P("x", None), P(None, None), P(None, None),
                      P(None, None), P(None, None), P(None, None)),
            out_specs=(P("x", None), P("x", None), P("x", None, None),
                       P("x", None, None)),
            check_vma=False,
        )
        out, enc, idx, loss = sharded(inp_flat, W_in, b_in2, embeddings,
                                      W_out, b_out2)
    else:
        out, enc, idx, loss = _shard_body(inp_flat, W_in, b_in2, embeddings,
                                          W_out, b_out2)

    encoding_indices = idx.reshape(B, T)
    return (out.reshape(B, T, INPUT_DIM), enc, encoding_indices,
            jnp.sum(loss))


# speedup vs baseline: 1.2716x; 1.2716x over previous
import functools

import jax
import jax.numpy as jnp
from jax.experimental import pallas as pl
from jax.experimental.pallas import tpu as pltpu
from jax.experimental.pallas import tpu_sc as plsc
from jax.sharding import Mesh, PartitionSpec as P

try:
    from jax import shard_map as _shard_map
except ImportError:
    from jax.experimental.shard_map import shard_map as _shard_map

EMBED_DIM = 256
CODEBOOK = 8192
INPUT_DIM = 768
N_ROWS = 16 * 576
TILE = 256
GW = 128


def _prep_kernel(emb_ref, emb_n_ref, emb_t_ref):
    emb = emb_ref[...]
    norms = jnp.sqrt(jnp.sum(emb * emb, axis=0, keepdims=True))
    emb_n_ref[...] = (emb / (norms + 1e-12)).astype(jnp.bfloat16)
    emb_t_ref[...] = emb.T


def _vq_kernel(inp_ref, w_in_ref, b_in_ref, emb_n_ref,
               x_ref, enc_ref, idx_ref):
    x = jnp.dot(inp_ref[...].astype(jnp.bfloat16),
                w_in_ref[...].astype(jnp.bfloat16),
                preferred_element_type=jnp.float32) + b_in_ref[...]
    x_ref[...] = x
    norm = jnp.sqrt(jnp.sum(x * x, axis=1, keepdims=True))
    x_n = x / (norm + 1e-12)
    logits = jnp.dot(x_n.astype(jnp.bfloat16), emb_n_ref[...],
                     preferred_element_type=jnp.float32)

    SLAB = CODEBOOK // 4
    iota_s = jax.lax.broadcasted_iota(jnp.int32, (TILE, SLAB), 1)

    def slab_argmax(s):
        sl = logits[:, s * SLAB:(s + 1) * SLAB]
        m = jnp.max(sl, axis=1, keepdims=True)
        j = jnp.min(jnp.where(sl == m, iota_s, CODEBOOK), axis=1) + s * SLAB
        return m[:, 0], j

    accv, accj = slab_argmax(0)
    accv = accv.astype(jnp.bfloat16).astype(jnp.float32)
    for s in range(1, 4):
        v, j = slab_argmax(s)
        upd = (v > accv) | ((v == accv) & (j < accj))
        accv = jnp.where(upd, v.astype(jnp.bfloat16).astype(jnp.float32), accv)
        accj = jnp.where(upd, j, accj)
    idx = accj

    iota = jax.lax.broadcasted_iota(jnp.int32, (TILE, CODEBOOK), 1)
    enc_ref[...] = (iota == idx[:, None]).astype(jnp.float32)
    idx_ref[0, 0, :] = idx


def _out_kernel(q_ref, x_ref, w_out_ref, b_out_ref, out_ref, loss_ref):
    q = q_ref[...]
    out_ref[...] = jnp.dot(q.astype(jnp.bfloat16),
                           w_out_ref[...].astype(jnp.bfloat16),
                           preferred_element_type=jnp.float32) + b_out_ref[...]
    diff = q - x_ref[...]
    part = 2.0 * jnp.sum(diff * diff) / (N_ROWS * EMBED_DIM)
    loss_ref[...] = part.reshape(1, 1, 1)


def _sc_gather(emb_t, idx_flat):
    n_rows = idx_flat.shape[1]
    mesh = plsc.VectorSubcoreMesh(core_axis_name="core",
                                  subcore_axis_name="subcore")

    @functools.partial(
        pl.kernel,
        out_type=jax.ShapeDtypeStruct((n_rows, EMBED_DIM), jnp.float32),
        mesh=mesh)
    def gather_kernel(t_hbm, i_hbm, q_hbm):
        def body(i_vmem, o_vmem):
            pltpu.sync_copy(t_hbm.at[i_vmem.at[0]], o_vmem)

        pltpu.emit_pipeline(
            body,
            grid=(n_rows // GW,),
            in_specs=[pl.BlockSpec((1, GW), index_map=lambda i: (0, i))],
            out_specs=[pl.BlockSpec((GW, EMBED_DIM), index_map=lambda i: (i, 0))],
            core_axis_name="subcore",
            dimension_semantics=(pltpu.PARALLEL,),
        )(i_hbm, q_hbm)

    return gather_kernel(emb_t, idx_flat)


def _shard_body(inp_flat, W_in, b_in, embeddings, W_out, b_out):
    n_rows = inp_flat.shape[0]
    n_tiles = n_rows // TILE

    emb_n_bf, emb_t = pl.pallas_call(
        _prep_kernel,
        in_specs=[pl.BlockSpec((EMBED_DIM, CODEBOOK), lambda: (0, 0))],
        out_specs=[pl.BlockSpec((EMBED_DIM, CODEBOOK), lambda: (0, 0)),
                   pl.BlockSpec((CODEBOOK, EMBED_DIM), lambda: (0, 0))],
        out_shape=[jax.ShapeDtypeStruct((EMBED_DIM, CODEBOOK), jnp.bfloat16),
                   jax.ShapeDtypeStruct((CODEBOOK, EMBED_DIM), jnp.float32)],
    )(embeddings)

    x, enc, idx = pl.pallas_call(
        _vq_kernel,
        grid=(n_tiles,),
        in_specs=[
            pl.BlockSpec((TILE, INPUT_DIM), lambda i: (i, 0)),
            pl.BlockSpec((INPUT_DIM, EMBED_DIM), lambda i: (0, 0)),
            pl.BlockSpec((1, EMBED_DIM), lambda i: (0, 0)),
            pl.BlockSpec((EMBED_DIM, CODEBOOK), lambda i: (0, 0)),
        ],
        out_specs=[
            pl.BlockSpec((TILE, EMBED_DIM), lambda i: (i, 0)),
            pl.BlockSpec((TILE, CODEBOOK), lambda i: (i, 0)),
            pl.BlockSpec((1, 1, TILE), lambda i: (i, 0, 0)),
        ],
        out_shape=[
            jax.ShapeDtypeStruct((n_rows, EMBED_DIM), jnp.float32),
            jax.ShapeDtypeStruct((n_rows, CODEBOOK), jnp.float32),
            jax.ShapeDtypeStruct((n_tiles, 1, TILE), jnp.int32),
        ],
        compiler_params=pltpu.CompilerParams(
            dimension_semantics=("arbitrary",)),
    )(inp_flat, W_in, b_in, emb_n_bf)

    q = _sc_gather(emb_t, idx.reshape(1, n_rows))

    out, loss = pl.pallas_call(
        _out_kernel,
        grid=(n_tiles,),
        in_specs=[
            pl.BlockSpec((TILE, EMBED_DIM), lambda i: (i, 0)),
            pl.BlockSpec((TILE, EMBED_DIM), lambda i: (i, 0)),
            pl.BlockSpec((EMBED_DIM, INPUT_DIM), lambda i: (0, 0)),
            pl.BlockSpec((1, INPUT_DIM), lambda i: (0, 0)),
        ],
        out_specs=[
            pl.BlockSpec((TILE, INPUT_DIM), lambda i: (i, 0)),
            pl.BlockSpec((1, 1, 1), lambda i: (i, 0, 0)),
        ],
        out_shape=[
            jax.ShapeDtypeStruct((n_rows, INPUT_DIM), jnp.float32),
            jax.ShapeDtypeStruct((n_tiles, 1, 1), jnp.float32),
        ],
        compiler_params=pltpu.CompilerParams(
            dimension_semantics=("arbitrary",)),
    )(q, x, W_out, b_out)

    return out, enc, idx, loss


@jax.jit
def kernel(inputs, W_in, b_in, embeddings, W_out, b_out):
    B, T, _ = inputs.shape
    inp_flat = inputs.reshape(N_ROWS, INPUT_DIM)
    b_in2 = b_in.reshape(1, EMBED_DIM)
    b_out2 = b_out.reshape(1, INPUT_DIM)

    devs = [d for d in jax.devices() if d.platform == "tpu"][:2]
    if len(devs) == 2:
        mesh = Mesh(devs, ("x",))
        sharded = _shard_map(
            _shard_body, mesh=mesh,
            in_specs=(P("x", None), P(None, None), P(None, None),
                      P(None, None), P(None, None), P(None, None)),
            out_specs=(P("x", None), P("x", None), P("x", None, None),
                       P("x", None, None)),
            check_vma=False,
        )
        out, enc, idx, loss = sharded(inp_flat, W_in, b_in2, embeddings,
                                      W_out, b_out2)
    else:
        out, enc, idx, loss = _shard_body(inp_flat, W_in, b_in2, embeddings,
                                          W_out, b_out2)

    encoding_indices = idx.reshape(B, T)
    return (out.reshape(B, T, INPUT_DIM), enc, encoding_indices,
            jnp.sum(loss))
